# baseline (speedup 1.0000x reference)
import numpy as np

import jax
import jax.numpy as jnp
from jax.experimental import pallas as pl

EPS = 1e-5


def _masks9(hw, cols):
    idx = jax.lax.broadcasted_iota(jnp.int32, (1, cols), 1)
    np_ = hw * hw
    w = idx % hw
    h = (idx % np_) // hw
    out = []
    for k in range(9):
        dy, dx = k // 3 - 1, k % 3 - 1
        hh = h + dy
        ww = w + dx
        out.append((hh >= 0) & (hh < hw) & (ww >= 0) & (ww < hw))
    return out

def _conv3c(x, w9, hw, masks):
    ci, cols = x.shape
    pad = jnp.zeros((ci, 32), x.dtype)
    xp = jnp.concatenate([pad, x, pad], axis=1)
    acc = None
    for k in range(9):
        dy, dx = k // 3 - 1, k % 3 - 1
        o = dy * hw + dx
        sl = xp[:, 32 + o:32 + o + cols]
        sl = jnp.where(masks[k], sl, 0.0)
        d = jax.lax.dot_general(w9[k], sl, (((1,), (0,)), ((), ())),
                                preferred_element_type=jnp.float32)
        acc = d if acc is None else acc + d
    return acc

def _maxpool_c(x, hw, tbi, sel):
    cols = x.shape[1]
    np_in = hw * hw
    np_out = np_in // 4
    pad = jnp.zeros((x.shape[0], 32), x.dtype)
    xp = jnp.concatenate([x, pad], axis=1)
    xm = jnp.maximum(x, xp[:, 1:1 + cols])
    xmp = jnp.concatenate([xm, pad], axis=1)
    xm2 = jnp.maximum(xm, xmp[:, hw:hw + cols])
    outs = []
    for i in range(tbi):
        outs.append(jax.lax.dot_general(
            xm2[:, i * np_in:(i + 1) * np_in], sel,
            (((1,), (0,)), ((), ())),
            preferred_element_type=jnp.float32))
    return jnp.concatenate(outs, axis=1)

def _interleave(z, hw, tbi, tmat):
    np_in = hw * hw
    outs = []
    for i in range(tbi):
        zc = jnp.concatenate(
            [zk[:, i * np_in:(i + 1) * np_in] for zk in z], axis=1)
        outs.append(jax.lax.dot_general(
            zc, tmat, (((1,), (0,)), ((), ())),
            preferred_element_type=jnp.float32))
    return jnp.concatenate(outs, axis=1)

def _stats_c(v):
    return jnp.stack([jnp.sum(v, axis=1), jnp.sum(v * v, axis=1)], axis=1)

def _acc(ref, st):
    i = pl.program_id(0)

    @pl.when(i == 0)
    def _():
        ref[...] = st

    @pl.when(i > 0)
    def _():
        ref[...] = ref[...] + st

def _affine_c(st, gb, n):
    m = st[:, 0] / n
    v = st[:, 1] / n - m * m
    inv = gb[:, 0] * jax.lax.rsqrt(v + EPS)
    return inv[:, None], (gb[:, 1] - m * inv)[:, None]

def _bmm(a, b, ca, cb):
    return jax.lax.dot_general(
        a, b, (((ca,), (cb,)), ((0,), (0,))),
        preferred_element_type=jnp.float32)

def _dgc(a, b, ca, cb):
    return jax.lax.dot_general(
        a, b, (((ca,), (cb,)), ((), ())),
        preferred_element_type=jnp.float32)


def _k1(x_ref, w3_ref, wds_ref, t1_ref, tds_ref, st1_ref, stds_ref):
    x = x_ref[...]
    cols = x.shape[1]
    masks = _masks9(28, cols)
    pad = jnp.zeros((1, 32), x.dtype)
    xp = jnp.concatenate([pad, x, pad], axis=1)
    t1 = None
    for k in range(9):
        dy, dx = k // 3 - 1, k % 3 - 1
        o = dy * 28 + dx
        sl = jnp.where(masks[k], xp[:, 32 + o:32 + o + cols], 0.0)
        term = w3_ref[...][:, k:k + 1] * sl
        t1 = term if t1 is None else t1 + term
    tds = wds_ref[...] * x
    t1_ref[...] = t1
    tds_ref[...] = tds
    _acc(st1_ref, _stats_c(t1))
    _acc(stds_ref, _stats_c(tds))


def _mid_kernel(n, hw):
    def kfn(t1_ref, st1_ref, gb_ref, w_ref, t2_ref, st2_ref):
        inv, sh = _affine_c(st1_ref[...], gb_ref[...], n)
        a = jnp.maximum(t1_ref[...] * inv + sh, 0.0)
        masks = _masks9(hw, a.shape[1])
        t2 = _conv3c(a, w_ref[...], hw, masks)
        t2_ref[...] = t2
        _acc(st2_ref, _stats_c(t2))
    return kfn


def _k3_f(n):
    def kfn(t2_ref, tds_ref, st2_ref, stds_ref, gb2_ref, gbds_ref,
            w3n_ref, wdsn_ref, sel_ref,
            skip1_ref, t1n_ref, tdsn_ref, st1n_ref, stdsn_ref):
        inv2, sh2 = _affine_c(st2_ref[...], gb2_ref[...], n)
        invd, shd = _affine_c(stds_ref[...], gbds_ref[...], n)
        s1 = jnp.maximum(
            t2_ref[...] * inv2 + sh2 + tds_ref[...] * invd + shd, 0.0)
        skip1_ref[...] = s1
        xp = _maxpool_c(s1, 28, 32, sel_ref[...])
        masks = _masks9(14, xp.shape[1])
        t1n = _conv3c(xp, w3n_ref[...], 14, masks)
        tdsn = _dgc(wdsn_ref[...], xp, 1, 0)
        t1n_ref[...] = t1n
        tdsn_ref[...] = tdsn
        _acc(st1n_ref, _stats_c(t1n))
        _acc(stdsn_ref, _stats_c(tdsn))
    return kfn


def _k5_f(n):
    def kfn(t2_ref, tds_ref, st2_ref, stds_ref, gb2_ref, gbds_ref,
            prew_ref, preb_ref, sel_ref,
            skip2_ref, xd_ref, stxd_ref):
        inv2, sh2 = _affine_c(st2_ref[...], gb2_ref[...], n)
        invd, shd = _affine_c(stds_ref[...], gbds_ref[...], n)
        s2 = jnp.maximum(
            t2_ref[...] * inv2 + sh2 + tds_ref[...] * invd + shd, 0.0)
        skip2_ref[...] = s2
        xp = _maxpool_c(s2, 14, 32, sel_ref[...])
        xd = _dgc(xp, prew_ref[...], 0, 0) + preb_ref[...]
        xd_ref[...] = xd
        _acc(stxd_ref, jnp.stack(
            [jnp.sum(xd, axis=0), jnp.sum(xd * xd, axis=0)], axis=0))
    return kfn


def _k6_f(n):
    def kfn(xd_ref, stxd_ref, gb_ref, qkvw_ref, qkvb_ref, outw_ref,
            outb_ref, m1w_ref, m1b_ref, m2w_ref, m2b_ref, postw_ref,
            postb_ref, upw_ref, tmat_ref, y1_ref, sty1_ref):
        rows = xd_ref.shape[0]
        tb = rows // 49
        st = stxd_ref[...]
        m = st[0] / n
        v = st[1] / n - m * m
        inv = gb_ref[0] * jax.lax.rsqrt(v + EPS)
        sh = gb_ref[1] - m * inv
        t = xd_ref[...] * inv + sh
        qkv = (jnp.dot(t, qkvw_ref[...], preferred_element_type=jnp.float32)
               + qkvb_ref[...]).reshape(tb, 49, 384)
        scale = np.float32(1.0 / np.sqrt(32.0))
        heads = []
        for h in range(4):
            qh = qkv[:, :, h * 32:(h + 1) * 32]
            kh = qkv[:, :, 128 + h * 32:128 + (h + 1) * 32]
            vh = qkv[:, :, 256 + h * 32:256 + (h + 1) * 32]
            s = _bmm(qh, kh, 2, 2) * scale
            mx = jnp.max(s, axis=-1, keepdims=True)
            e = jnp.exp(s - mx)
            p = e / jnp.sum(e, axis=-1, keepdims=True)
            heads.append(_bmm(p, vh, 2, 1))
        o = jnp.concatenate(heads, axis=2).reshape(rows, 128)
        o = jnp.dot(o, outw_ref[...], preferred_element_type=jnp.float32) \
            + outb_ref[...]
        x2 = t + o
        hm = jnp.dot(x2, m1w_ref[...], preferred_element_type=jnp.float32) \
            + m1b_ref[...]
        hm = 0.5 * hm * (1.0 + jax.lax.erf(hm * np.float32(1.0 / np.sqrt(2.0))))
        h2 = jnp.dot(hm, m2w_ref[...], preferred_element_type=jnp.float32) \
            + m2b_ref[...]
        xa = x2 + h2
        xo = jnp.dot(xa, postw_ref[...], preferred_element_type=jnp.float32) \
            + postb_ref[...]
        upw = upw_ref[...]
        z = [_dgc(upw[i], xo, 0, 1) for i in range(4)]
        y1 = _interleave(z, 7, tb, tmat_ref[...])
        y1_ref[...] = y1
        _acc(sty1_ref, _stats_c(y1))
    return kfn


def _merge_kernel(n, hw):
    def kfn(y_ref, sty_ref, gby_ref, skip_ref,
            wc1a_ref, wc1b_ref, wdsa_ref, wdsb_ref,
            t1_ref, tds_ref, st1_ref, stds_ref):
        inv, sh = _affine_c(sty_ref[...], gby_ref[...], n)
        yb = jnp.maximum(y_ref[...] * inv + sh, 0.0)
        sk = skip_ref[...]
        masks = _masks9(hw, yb.shape[1])
        t1 = _conv3c(yb, wc1a_ref[...], hw, masks) \
            + _conv3c(sk, wc1b_ref[...], hw, masks)
        tds = _dgc(wdsa_ref[...], yb, 1, 0) + _dgc(wdsb_ref[...], sk, 1, 0)
        t1_ref[...] = t1
        tds_ref[...] = tds
        _acc(st1_ref, _stats_c(t1))
        _acc(stds_ref, _stats_c(tds))
    return kfn


def _k9_f(n):
    def kfn(t2_ref, tds_ref, st2_ref, stds_ref, gb2_ref, gbds_ref, upw_ref,
            tmat_ref, y0_ref, sty0_ref):
        inv2, sh2 = _affine_c(st2_ref[...], gb2_ref[...], n)
        invd, shd = _affine_c(stds_ref[...], gbds_ref[...], n)
        d = jnp.maximum(
            t2_ref[...] * inv2 + sh2 + tds_ref[...] * invd + shd, 0.0)
        upw = upw_ref[...]
        z = [_dgc(upw[i], d, 0, 0) for i in range(4)]
        y0 = _interleave(z, 14, 32, tmat_ref[...])
        y0_ref[...] = y0
        _acc(sty0_ref, _stats_c(y0))
    return kfn


def _k12_f(n):
    def kfn(t2_ref, tds_ref, st2_ref, stds_ref, gb2_ref, gbds_ref,
            mmat_ref, hw_ref, hb_ref, out_ref):
        inv2, sh2 = _affine_c(st2_ref[...], gb2_ref[...], n)
        invd, shd = _affine_c(stds_ref[...], gbds_ref[...], n)
        d = jnp.maximum(
            t2_ref[...] * inv2 + sh2 + tds_ref[...] * invd + shd, 0.0)
        feat = _dgc(d, mmat_ref[...], 1, 0)
        logits = _dgc(feat, hw_ref[...], 0, 0) + hb_ref[...]
        mx = jnp.max(logits, axis=-1, keepdims=True)
        l = logits - mx
        out_ref[...] = l - jnp.log(jnp.sum(jnp.exp(l), axis=-1, keepdims=True))
    return kfn


def _w3t(w):
    return jnp.transpose(w, (2, 3, 0, 1)).reshape(9, w.shape[0], w.shape[1])

def _w1t(w):
    return w[:, :, 0, 0]

def _wupt(w):
    return jnp.transpose(w, (2, 3, 0, 1)).reshape(4, w.shape[0], w.shape[1])

def _gbc(g, b):
    return jnp.stack([g, b], axis=1)

def _rep(a):
    nd = a.ndim
    return pl.BlockSpec(a.shape, lambda i, _nd=nd: (0,) * _nd)

def _col(c, cols):
    return pl.BlockSpec((c, cols), lambda i: (0, i))

def _row(rows, c):
    return pl.BlockSpec((rows, c), lambda i: (i, 0))

def _st2(c):
    return pl.BlockSpec((c, 2), lambda i: (0, 0))

def _sds(shape):
    return jax.ShapeDtypeStruct(shape, jnp.float32)


def kernel(x, params):
    B = x.shape[0]
    p = params
    N28, N14 = B * 784, B * 196
    C28s, C28l = 8 * 784, 32 * 784
    C14 = 32 * 196
    G28, G32 = B // 8, B // 32

    def call(fn, grid, ins, in_specs, out_shapes, out_specs):
        return pl.pallas_call(
            fn, grid=(grid,), in_specs=in_specs, out_specs=out_specs,
            out_shape=out_shapes)(*ins)

    xf = x.reshape(1, B * 784)

    def selmat(hw):
        h2 = hw // 2
        n_in, n_out = hw * hw, h2 * h2
        q = np.arange(n_out)
        pos = (2 * (q // h2)) * hw + 2 * (q % h2)
        s = np.zeros((n_in, n_out), np.float32)
        s[pos, q] = 1.0
        return jnp.asarray(s)

    def tmat(hw):
        n_in = hw * hw
        t = np.zeros((4 * n_in, 4 * n_in), np.float32)
        for k in range(2):
            for l in range(2):
                m = np.arange(n_in)
                dst = (2 * (m // hw) + k) * (2 * hw) + 2 * (m % hw) + l
                t[(2 * k + l) * n_in + m, dst] = 1.0
        return jnp.asarray(t)

    sel28 = selmat(28)
    sel14 = selmat(14)
    tm7 = tmat(7)
    tm14 = tmat(14)
    mmat = jnp.asarray(
        (np.arange(32 * 784)[:, None] // 784 == np.arange(32)[None, :])
        .astype(np.float32) / 784.0)

    e1 = p['enc1']
    w3a = _w3t(e1['c1_w'])[:, :, 0].T
    wds = _w1t(e1['ds_w'])
    t1, tds, st1, stds = call(
        _k1, G28, [xf, w3a, wds],
        [pl.BlockSpec((1, C28s), lambda i: (0, i)), _rep(w3a), _rep(wds)],
        [_sds((32, N28)), _sds((32, N28)), _sds((32, 2)), _sds((32, 2))],
        [_col(32, C28s), _col(32, C28s), _st2(32), _st2(32)])

    w2 = _w3t(e1['c2_w'])
    gb1 = _gbc(e1['c1_g'], e1['c1_b'])
    t2, st2 = call(
        _mid_kernel(N28, 28), G28, [t1, st1, gb1, w2],
        [_col(32, C28s), _st2(32), _rep(gb1), _rep(w2)],
        [_sds((32, N28)), _sds((32, 2))],
        [_col(32, C28s), _st2(32)])

    e2 = p['enc2']
    w3n = _w3t(e2['c1_w'])
    wdsn = _w1t(e2['ds_w'])
    gb2 = _gbc(e1['c2_g'], e1['c2_b'])
    gbds = _gbc(e1['ds_g'], e1['ds_b'])
    skip1, t1b, tdsb, st1b, stdsb = call(
        _k3_f(N28), G32, [t2, tds, st2, stds, gb2, gbds, w3n, wdsn, sel28],
        [_col(32, C28l), _col(32, C28l), _st2(32), _st2(32),
         _rep(gb2), _rep(gbds), _rep(w3n), _rep(wdsn), _rep(sel28)],
        [_sds((32, N28)), _sds((64, N14)), _sds((64, N14)),
         _sds((64, 2)), _sds((64, 2))],
        [_col(32, C28l), _col(64, C14), _col(64, C14),
         _st2(64), _st2(64)])

    w2b = _w3t(e2['c2_w'])
    gb1b = _gbc(e2['c1_g'], e2['c1_b'])
    t2b, st2b = call(
        _mid_kernel(N14, 14), G32, [t1b, st1b, gb1b, w2b],
        [_col(64, C14), _st2(64), _rep(gb1b), _rep(w2b)],
        [_sds((64, N14)), _sds((64, 2))],
        [_col(64, C14), _st2(64)])

    prew = _w1t(p['pre_w']).T
    preb = p['pre_b']
    gb2b = _gbc(e2['c2_g'], e2['c2_b'])
    gbdsb = _gbc(e2['ds_g'], e2['ds_b'])
    skip2, xd, stxd = call(
        _k5_f(N14), G32,
        [t2b, tdsb, st2b, stdsb, gb2b, gbdsb, prew, preb, sel14],
        [_col(64, C14), _col(64, C14), _st2(64), _st2(64),
         _rep(gb2b), _rep(gbdsb), _rep(prew), _rep(preb), _rep(sel14)],
        [_sds((64, N14)), _sds((B * 49, 128)), _sds((2, 128))],
        [_col(64, C14), _row(32 * 49, 128),
         pl.BlockSpec((2, 128), lambda i: (0, 0))])

    at = p['attn']
    gbat = jnp.stack([at['bn_g'], at['bn_b']])
    qkvw = at['qkv_w'].T
    outw = at['out_w'].T
    m1w = _w1t(at['m1_w']).T
    m2w = _w1t(at['m2_w']).T
    postw = _w1t(p['post_w']).T
    upw1 = _wupt(p['up1']['w'])
    y1, sty1 = call(
        _k6_f(B * 49), G32,
        [xd, stxd, gbat, qkvw, at['qkv_b'], outw, at['out_b'],
         m1w, at['m1_b'], m2w, at['m2_b'], postw, p['post_b'], upw1, tm7],
        [_row(32 * 49, 128), pl.BlockSpec((2, 128), lambda i: (0, 0)),
         _rep(gbat), _rep(qkvw), _rep(at['qkv_b']), _rep(outw),
         _rep(at['out_b']), _rep(m1w), _rep(at['m1_b']), _rep(m2w),
         _rep(at['m2_b']), _rep(postw), _rep(p['post_b']), _rep(upw1),
         _rep(tm7)],
        [_sds((64, N14)), _sds((64, 2))],
        [_col(64, C14), _st2(64)])

    d1 = p['dec1']
    wc1 = _w3t(d1['c1_w'])
    wc1a, wc1b = wc1[:, :, :64], wc1[:, :, 64:]
    wds1 = _w1t(d1['ds_w'])
    wdsa, wdsb = wds1[:, :64], wds1[:, 64:]
    gbu1 = _gbc(p['up1']['g'], p['up1']['b'])
    t1d, tdsd, st1d, stdsd = call(
        _merge_kernel(N14, 14), G32,
        [y1, sty1, gbu1, skip2, wc1a, wc1b, wdsa, wdsb],
        [_col(64, C14), _st2(64), _rep(gbu1), _col(64, C14),
         _rep(wc1a), _rep(wc1b), _rep(wdsa), _rep(wdsb)],
        [_sds((64, N14)), _sds((64, N14)), _sds((64, 2)), _sds((64, 2))],
        [_col(64, C14), _col(64, C14), _st2(64), _st2(64)])

    w2d = _w3t(d1['c2_w'])
    gb1d = _gbc(d1['c1_g'], d1['c1_b'])
    t2d, st2d = call(
        _mid_kernel(N14, 14), G32, [t1d, st1d, gb1d, w2d],
        [_col(64, C14), _st2(64), _rep(gb1d), _rep(w2d)],
        [_sds((64, N14)), _sds((64, 2))],
        [_col(64, C14), _st2(64)])

    upw0 = _wupt(p['up0']['w'])
    gb2d = _gbc(d1['c2_g'], d1['c2_b'])
    gbdsd = _gbc(d1['ds_g'], d1['ds_b'])
    y0, sty0 = call(
        _k9_f(N14), G32, [t2d, tdsd, st2d, stdsd, gb2d, gbdsd, upw0, tm14],
        [_col(64, C14), _col(64, C14), _st2(64), _st2(64),
         _rep(gb2d), _rep(gbdsd), _rep(upw0), _rep(tm14)],
        [_sds((32, N28)), _sds((32, 2))],
        [_col(32, C28l), _st2(32)])

    d0 = p['dec0']
    wc0 = _w3t(d0['c1_w'])
    wc0a, wc0b = wc0[:, :, :32], wc0[:, :, 32:]
    wds0 = _w1t(d0['ds_w'])
    wds0a, wds0b = wds0[:, :32], wds0[:, 32:]
    gbu0 = _gbc(p['up0']['g'], p['up0']['b'])
    t1e, tdse, st1e, stdse = call(
        _merge_kernel(N28, 28), G28,
        [y0, sty0, gbu0, skip1, wc0a, wc0b, wds0a, wds0b],
        [_col(32, C28s), _st2(32), _rep(gbu0), _col(32, C28s),
         _rep(wc0a), _rep(wc0b), _rep(wds0a), _rep(wds0b)],
        [_sds((32, N28)), _sds((32, N28)), _sds((32, 2)), _sds((32, 2))],
        [_col(32, C28s), _col(32, C28s), _st2(32), _st2(32)])

    w2e = _w3t(d0['c2_w'])
    gb1e = _gbc(d0['c1_g'], d0['c1_b'])
    t2e, st2e = call(
        _mid_kernel(N28, 28), G28, [t1e, st1e, gb1e, w2e],
        [_col(32, C28s), _st2(32), _rep(gb1e), _rep(w2e)],
        [_sds((32, N28)), _sds((32, 2))],
        [_col(32, C28s), _st2(32)])

    hw = p['head_w'].T
    gb2e = _gbc(d0['c2_g'], d0['c2_b'])
    gbdse = _gbc(d0['ds_g'], d0['ds_b'])
    out = call(
        _k12_f(N28), G32, [t2e, tdse, st2e, stdse, gb2e, gbdse,
                           mmat, hw, p['head_b']],
        [_col(32, C28l), _col(32, C28l), _st2(32), _st2(32),
         _rep(gb2e), _rep(gbdse), _rep(mmat), _rep(hw), _rep(p['head_b'])],
        [_sds((B, 10))],
        [_row(32, 10)])

    return out[0] if isinstance(out, (list, tuple)) else out
